# baseline (speedup 1.0000x reference)
import jax
import jax.numpy as jnp
from jax import lax
from jax.experimental import pallas as pl
from jax.experimental.pallas import tpu as pltpu
from jax.experimental.pallas import tpu_sc as plsc

N = 10000
E = 320000
D = 128
H = 256
OUT = 128

NC = 2
NS = 16
NW = NC * NS
EPW = E // NW
CHUNK = 48
NCHUNK = 208
TAIL = EPW - NCHUNK * CHUNK
RPT = 624
RREM = N - NS * RPT
L = 16


def _sc_body(ab, z, ga, gb, src, dst, srows, w2e, b2e,
             edge_out, zpart, spart,
             ga_v, gb_v, src_v, dst_v,
             arow, brow, zrow, srow, out_v, w2e_v, b2e_v,
             zacc, sacc):
    cid = lax.axis_index("c")
    sid = lax.axis_index("s")
    wid = cid * NS + sid
    lanes = lax.iota(jnp.int32, L)
    zero16 = jnp.zeros((L,), jnp.float32)

    def zr(i, c):
        for j in range(D // L):
            zrow[i, pl.ds(j * L, L)] = zero16
        srow[i, :] = zero16
        return c
    lax.fori_loop(0, CHUNK, zr, 0)
    for k in range(13):
        pltpu.sync_copy(zrow, zacc.at[pl.ds(sid * RPT + k * CHUNK, CHUNK)])
        pltpu.sync_copy(srow, sacc.at[pl.ds(sid * RPT + k * CHUNK, CHUNK)])

    @pl.when(sid == NS - 1)
    def _zero_rem():
        pltpu.sync_copy(zrow.at[pl.ds(0, RREM)],
                        zacc.at[pl.ds(NS * RPT, RREM)])
        pltpu.sync_copy(srow.at[pl.ds(0, RREM)],
                        sacc.at[pl.ds(NS * RPT, RREM)])

    pltpu.sync_copy(w2e, w2e_v)
    pltpu.sync_copy(b2e, b2e_v)

    plsc.subcore_barrier()

    def do_chunk(base, ne):
        pltpu.sync_copy(ga.at[pl.ds(base, ne)], ga_v.at[pl.ds(0, ne)])
        pltpu.sync_copy(gb.at[pl.ds(base, ne)], gb_v.at[pl.ds(0, ne)])
        pltpu.sync_copy(src.at[pl.ds(base, ne)], src_v.at[pl.ds(0, ne)])
        pltpu.sync_copy(dst.at[pl.ds(base, ne)], dst_v.at[pl.ds(0, ne)])
        pltpu.sync_copy(srows.at[pl.ds(base, ne)], srow.at[pl.ds(0, ne)])
        pltpu.sync_copy(ab.at[ga_v.at[pl.ds(0, ne)]], arow.at[pl.ds(0, ne)])
        pltpu.sync_copy(ab.at[gb_v.at[pl.ds(0, ne)]], brow.at[pl.ds(0, ne)])
        pltpu.sync_copy(z.at[src_v.at[pl.ds(0, ne)]], zrow.at[pl.ds(0, ne)])
        pltpu.sync_copy(zrow.at[pl.ds(0, ne)],
                        zacc.at[dst_v.at[pl.ds(0, ne)]], add=True)
        pltpu.sync_copy(srow.at[pl.ds(0, ne)],
                        sacc.at[dst_v.at[pl.ds(0, ne)]], add=True)

        for g in range(ne // L):
            e_vec = lanes + g * L

            def hchunk(hc, acc):
                w16 = w2e_v[pl.ds(hc * L, L)]
                for j in range(L):
                    h_vec = jnp.full((L,), hc * L + j, jnp.int32)
                    a = plsc.load_gather(arow, [e_vec, h_vec])
                    b = plsc.load_gather(brow, [e_vec, h_vec])
                    acc = acc + jnp.maximum(a + b, 0.0) * w16[j]
                return acc
            acc = lax.fori_loop(0, H // L, hchunk, zero16)
            out_v[pl.ds(g * L, L)] = acc + b2e_v[:]
        pltpu.sync_copy(out_v.at[pl.ds(0, ne)], edge_out.at[pl.ds(base, ne)])

    def chunk_body(ci, c):
        do_chunk(wid * EPW + ci * CHUNK, CHUNK)
        return c
    lax.fori_loop(0, NCHUNK, chunk_body, 0)
    do_chunk(wid * EPW + NCHUNK * CHUNK, TAIL)

    plsc.subcore_barrier()

    def _dump(rbase, rn):
        pltpu.sync_copy(zacc.at[pl.ds(rbase, rn)], zrow.at[pl.ds(0, rn)])
        pltpu.sync_copy(zrow.at[pl.ds(0, rn)], zpart.at[cid, pl.ds(rbase, rn)])
        pltpu.sync_copy(sacc.at[pl.ds(rbase, rn)], srow.at[pl.ds(0, rn)])
        pltpu.sync_copy(srow.at[pl.ds(0, rn)], spart.at[cid, pl.ds(rbase, rn)])

    for k in range(13):
        _dump(sid * RPT + k * CHUNK, CHUNK)

    @pl.when(sid == NS - 1)
    def _dump_rem():
        _dump(NS * RPT, RREM)


_sc_call = pl.kernel(
    _sc_body,
    out_type=(
        jax.ShapeDtypeStruct((E,), jnp.float32),
        jax.ShapeDtypeStruct((NC, N, D), jnp.float32),
        jax.ShapeDtypeStruct((NC, N, L), jnp.float32),
    ),
    mesh=plsc.VectorSubcoreMesh(core_axis_name="c", subcore_axis_name="s"),
    compiler_params=pltpu.CompilerParams(needs_layout_passes=False,
                                         use_tc_tiling_on_sc=False),
    scratch_types=[
        pltpu.VMEM((CHUNK,), jnp.int32),
        pltpu.VMEM((CHUNK,), jnp.int32),
        pltpu.VMEM((CHUNK,), jnp.int32),
        pltpu.VMEM((CHUNK,), jnp.int32),
        pltpu.VMEM((CHUNK, H), jnp.float32),
        pltpu.VMEM((CHUNK, H), jnp.float32),
        pltpu.VMEM((CHUNK, D), jnp.float32),
        pltpu.VMEM((CHUNK, L), jnp.float32),
        pltpu.VMEM((CHUNK,), jnp.float32),
        pltpu.VMEM((H,), jnp.float32),
        pltpu.VMEM((L,), jnp.float32),
        pltpu.VMEM_SHARED((N, D), jnp.float32),
        pltpu.VMEM_SHARED((N, L), jnp.float32),
    ],
)


def _mm_bias_body(x_ref, w_ref, b_ref, o_ref):
    o_ref[...] = (jnp.dot(x_ref[...], w_ref[...],
                          preferred_element_type=jnp.float32) + b_ref[...])


def _mm_bias(x, w, b, bn=1000):
    n, k = x.shape
    m = w.shape[1]
    return pl.pallas_call(
        _mm_bias_body,
        grid=(n // bn,),
        in_specs=[
            pl.BlockSpec((bn, k), lambda i: (i, 0)),
            pl.BlockSpec((k, m), lambda i: (0, 0)),
            pl.BlockSpec((1, m), lambda i: (0, 0)),
        ],
        out_specs=pl.BlockSpec((bn, m), lambda i: (i, 0)),
        out_shape=jax.ShapeDtypeStruct((n, m), jnp.float32),
    )(x, w, b.reshape(1, m))


def _attr_body(zp_ref, sp_ref, w1az_ref, w1aa_ref, b1a_ref, w2a_ref, b2a_ref,
               o_ref):
    zsum = zp_ref[0] + zp_ref[1]
    ssum = sp_ref[0] + sp_ref[1]
    cnt = jnp.maximum(ssum[:, 1:2], 1.0)
    attr_mean = ssum[:, 0:1] / cnt
    zmean = zsum / cnt
    h = jnp.dot(zmean, w1az_ref[...], preferred_element_type=jnp.float32)
    h = jnp.maximum(h + attr_mean * w1aa_ref[...] + b1a_ref[...], 0.0)
    o_ref[...] = (jnp.dot(h, w2a_ref[...],
                          preferred_element_type=jnp.float32) + b2a_ref[...])


def _attr_call(zpart, spart, w1az, w1aa, b1a, w2a, b2a, bn=1000):
    return pl.pallas_call(
        _attr_body,
        grid=(N // bn,),
        in_specs=[
            pl.BlockSpec((NC, bn, D), lambda i: (0, i, 0)),
            pl.BlockSpec((NC, bn, L), lambda i: (0, i, 0)),
            pl.BlockSpec((D, H), lambda i: (0, 0)),
            pl.BlockSpec((1, H), lambda i: (0, 0)),
            pl.BlockSpec((1, H), lambda i: (0, 0)),
            pl.BlockSpec((H, OUT), lambda i: (0, 0)),
            pl.BlockSpec((1, OUT), lambda i: (0, 0)),
        ],
        out_specs=pl.BlockSpec((bn, OUT), lambda i: (i, 0)),
        out_shape=jax.ShapeDtypeStruct((N, OUT), jnp.float32),
    )(zpart, spart, w1az, w1aa, b1a, w2a, b2a)


def kernel(z, edge_index, edge_attr, W1a, b1a, W2a, b2a, W1e, b1e, W2e, b2e):
    src = edge_index[0].astype(jnp.int32)
    dst = edge_index[1].astype(jnp.int32)
    attr = edge_attr[:, 0]
    ga = src * 2
    gb = dst * 2 + 1
    srows = jnp.concatenate(
        [attr[:, None], jnp.ones((E, 1), jnp.float32),
         jnp.zeros((E, L - 2), jnp.float32)], axis=1)
    wcat = jnp.concatenate([W1e[:D], W1e[D:]], axis=1)
    bcat = jnp.concatenate([b1e, jnp.zeros((H,), b1e.dtype)])
    ab = _mm_bias(z, wcat, bcat)
    ab2 = ab.reshape(2 * N, H)
    w2e = W2e[:, 0]
    b2e16 = jnp.broadcast_to(b2e, (L,))
    edge_out, zpart, spart = _sc_call(ab2, z, ga, gb, src, dst, srows,
                                      w2e, b2e16)
    x_recon = _attr_call(zpart, spart, W1a[1:], W1a[0:1],
                         b1a.reshape(1, H), W2a, b2a.reshape(1, OUT))
    return (x_recon, edge_out.reshape(E, 1))


# speedup vs baseline: 1.0733x; 1.0733x over previous
import jax
import jax.numpy as jnp
from jax import lax
from jax.experimental import pallas as pl
from jax.experimental.pallas import tpu as pltpu
from jax.experimental.pallas import tpu_sc as plsc

N = 10000
E = 320000
D = 128
H = 256
OUT = 128
HH = H // 2

NC = 2
NS = 16
NW = NC * NS
EPW = E // NW
CHUNK = 96
NCHUNK = 105
EPAD = NCHUNK * CHUNK
PAD = EPAD - EPW
NROW = N + 8
RPT = 624
L = 16

RCOPIES = tuple((k * CHUNK, CHUNK) for k in range(6)) + ((576, 48),)


def _sc_body(ab_lo, ab_hi, z, idxp, srows, w2e, b2e,
             edge_out, zpart, spart,
             idxv, abrow, zrow, srow, out_v, w2e_v, b2e_v,
             zacc, sacc):
    cid = lax.axis_index("c")
    sid = lax.axis_index("s")
    wid = cid * NS + sid
    lanes = lax.iota(jnp.int32, L)
    zero16 = jnp.zeros((L,), jnp.float32)

    def zr(i, c):
        for j in range(D // L):
            zrow[i, pl.ds(j * L, L)] = zero16
        srow[i, :] = zero16
        return c
    lax.fori_loop(0, CHUNK, zr, 0)
    for off, sz in RCOPIES:
        pltpu.sync_copy(zrow.at[pl.ds(0, sz)],
                        zacc.at[pl.ds(sid * RPT + off, sz)])
        pltpu.sync_copy(srow.at[pl.ds(0, sz)],
                        sacc.at[pl.ds(sid * RPT + off, sz)])

    @pl.when(sid == NS - 1)
    def _zero_rem():
        pltpu.sync_copy(zrow.at[pl.ds(0, NROW - NS * RPT)],
                        zacc.at[pl.ds(NS * RPT, NROW - NS * RPT)])
        pltpu.sync_copy(srow.at[pl.ds(0, NROW - NS * RPT)],
                        sacc.at[pl.ds(NS * RPT, NROW - NS * RPT)])

    pltpu.sync_copy(w2e, w2e_v)
    pltpu.sync_copy(b2e, b2e_v)

    plsc.subcore_barrier()

    def chunk_body(ci, c):
        ib = (wid * NCHUNK + ci) * (4 * CHUNK)
        eb = (wid * NCHUNK + ci) * CHUNK
        pltpu.sync_copy(idxp.at[pl.ds(ib, 4 * CHUNK)], idxv)
        pltpu.sync_copy(srows.at[pl.ds(eb, CHUNK)], srow)
        pltpu.sync_copy(z.at[idxv.at[pl.ds(2 * CHUNK, CHUNK)]], zrow)
        pltpu.sync_copy(zrow, zacc.at[idxv.at[pl.ds(3 * CHUNK, CHUNK)]],
                        add=True)
        pltpu.sync_copy(srow, sacc.at[idxv.at[pl.ds(3 * CHUNK, CHUNK)]],
                        add=True)

        for abh, first in ((ab_lo, True), (ab_hi, False)):
            pltpu.sync_copy(abh.at[idxv.at[pl.ds(0, 2 * CHUNK)]], abrow)
            wbase = 0 if first else HH
            for g in range(CHUNK // L):
                e_vec = lanes + g * L

                def hchunk(hc, acc):
                    w16 = w2e_v[pl.ds(wbase + hc * L, L)]
                    for j in range(L):
                        h_vec = jnp.full((L,), hc * L + j, jnp.int32)
                        a = plsc.load_gather(abrow, [e_vec, h_vec])
                        b = plsc.load_gather(abrow, [e_vec + CHUNK, h_vec])
                        acc = acc + jnp.maximum(a + b, 0.0) * w16[j]
                    return acc
                acc = lax.fori_loop(0, HH // L, hchunk, zero16)
                if first:
                    out_v[pl.ds(g * L, L)] = acc + b2e_v[:]
                else:
                    out_v[pl.ds(g * L, L)] = out_v[pl.ds(g * L, L)] + acc
        pltpu.sync_copy(out_v, edge_out.at[pl.ds(eb, CHUNK)])
        return c

    lax.fori_loop(0, NCHUNK, chunk_body, 0)

    plsc.subcore_barrier()

    def _dump(rbase, rn_static):
        pltpu.sync_copy(zacc.at[pl.ds(rbase, rn_static)],
                        zrow.at[pl.ds(0, rn_static)])
        pltpu.sync_copy(zrow.at[pl.ds(0, rn_static)],
                        zpart.at[cid, pl.ds(rbase, rn_static)])
        pltpu.sync_copy(sacc.at[pl.ds(rbase, rn_static)],
                        srow.at[pl.ds(0, rn_static)])
        pltpu.sync_copy(srow.at[pl.ds(0, rn_static)],
                        spart.at[cid, pl.ds(rbase, rn_static)])

    for off, sz in RCOPIES:
        _dump(sid * RPT + off, sz)

    @pl.when(sid == NS - 1)
    def _dump_rem():
        _dump(NS * RPT, N - NS * RPT)


_sc_call = pl.kernel(
    _sc_body,
    out_type=(
        jax.ShapeDtypeStruct((NW * EPAD,), jnp.float32),
        jax.ShapeDtypeStruct((NC, N, D), jnp.float32),
        jax.ShapeDtypeStruct((NC, N, L), jnp.float32),
    ),
    mesh=plsc.VectorSubcoreMesh(core_axis_name="c", subcore_axis_name="s"),
    compiler_params=pltpu.CompilerParams(needs_layout_passes=False,
                                         use_tc_tiling_on_sc=False),
    scratch_types=[
        pltpu.VMEM((4 * CHUNK,), jnp.int32),
        pltpu.VMEM((2 * CHUNK, HH), jnp.float32),
        pltpu.VMEM((CHUNK, D), jnp.float32),
        pltpu.VMEM((CHUNK, L), jnp.float32),
        pltpu.VMEM((CHUNK,), jnp.float32),
        pltpu.VMEM((H,), jnp.float32),
        pltpu.VMEM((L,), jnp.float32),
        pltpu.VMEM_SHARED((NROW, D), jnp.float32),
        pltpu.VMEM_SHARED((NROW, L), jnp.float32),
    ],
)


def _mm_bias_body(x_ref, w_ref, b_ref, o_ref):
    o_ref[...] = (jnp.dot(x_ref[...], w_ref[...],
                          preferred_element_type=jnp.float32) + b_ref[...])


def _mm_bias(x, w, b, bn=1000):
    n, k = x.shape
    m = w.shape[1]
    return pl.pallas_call(
        _mm_bias_body,
        grid=(n // bn,),
        in_specs=[
            pl.BlockSpec((bn, k), lambda i: (i, 0)),
            pl.BlockSpec((k, m), lambda i: (0, 0)),
            pl.BlockSpec((1, m), lambda i: (0, 0)),
        ],
        out_specs=pl.BlockSpec((bn, m), lambda i: (i, 0)),
        out_shape=jax.ShapeDtypeStruct((n, m), jnp.float32),
    )(x, w, b.reshape(1, m))


def _attr_body(zp_ref, sp_ref, w1az_ref, w1aa_ref, b1a_ref, w2a_ref, b2a_ref,
               o_ref):
    zsum = zp_ref[0] + zp_ref[1]
    ssum = sp_ref[0] + sp_ref[1]
    cnt = jnp.maximum(ssum[:, 1:2], 1.0)
    attr_mean = ssum[:, 0:1] / cnt
    zmean = zsum / cnt
    h = jnp.dot(zmean, w1az_ref[...], preferred_element_type=jnp.float32)
    h = jnp.maximum(h + attr_mean * w1aa_ref[...] + b1a_ref[...], 0.0)
    o_ref[...] = (jnp.dot(h, w2a_ref[...],
                          preferred_element_type=jnp.float32) + b2a_ref[...])


def _attr_call(zpart, spart, w1az, w1aa, b1a, w2a, b2a, bn=1000):
    return pl.pallas_call(
        _attr_body,
        grid=(N // bn,),
        in_specs=[
            pl.BlockSpec((NC, bn, D), lambda i: (0, i, 0)),
            pl.BlockSpec((NC, bn, L), lambda i: (0, i, 0)),
            pl.BlockSpec((D, H), lambda i: (0, 0)),
            pl.BlockSpec((1, H), lambda i: (0, 0)),
            pl.BlockSpec((1, H), lambda i: (0, 0)),
            pl.BlockSpec((H, OUT), lambda i: (0, 0)),
            pl.BlockSpec((1, OUT), lambda i: (0, 0)),
        ],
        out_specs=pl.BlockSpec((bn, OUT), lambda i: (i, 0)),
        out_shape=jax.ShapeDtypeStruct((N, OUT), jnp.float32),
    )(zpart, spart, w1az, w1aa, b1a, w2a, b2a)


def _pad_tile(x, fill):
    x2 = x.reshape(NW, EPW)
    pad = jnp.full((NW, PAD), fill, x.dtype)
    return jnp.concatenate([x2, pad], axis=1)


def kernel(z, edge_index, edge_attr, W1a, b1a, W2a, b2a, W1e, b1e, W2e, b2e):
    src = edge_index[0].astype(jnp.int32)
    dst = edge_index[1].astype(jnp.int32)
    attr = edge_attr[:, 0]
    ga_p = _pad_tile(src * 2, 0).reshape(NW, NCHUNK, CHUNK)
    gb_p = _pad_tile(dst * 2 + 1, 1).reshape(NW, NCHUNK, CHUNK)
    src_p = _pad_tile(src, 0).reshape(NW, NCHUNK, CHUNK)
    dst_p = _pad_tile(dst, N).reshape(NW, NCHUNK, CHUNK)
    idxp = jnp.stack([ga_p, gb_p, src_p, dst_p], axis=2).reshape(-1)
    srows = jnp.concatenate(
        [attr[:, None], jnp.ones((E, 1), jnp.float32),
         jnp.zeros((E, L - 2), jnp.float32)], axis=1)
    srows_p = jnp.concatenate(
        [srows.reshape(NW, EPW, L), jnp.zeros((NW, PAD, L), jnp.float32)],
        axis=1).reshape(NW * EPAD, L)
    wcat = jnp.concatenate(
        [W1e[:D, :HH], W1e[D:, :HH], W1e[:D, HH:], W1e[D:, HH:]], axis=1)
    bcat = jnp.concatenate(
        [b1e[:HH], jnp.zeros((HH,), b1e.dtype),
         b1e[HH:], jnp.zeros((HH,), b1e.dtype)])
    ab = _mm_bias(z, wcat, bcat)
    ab_lo = ab[:, :H].reshape(2 * N, HH)
    ab_hi = ab[:, H:].reshape(2 * N, HH)
    w2e = W2e[:, 0]
    b2e16 = jnp.broadcast_to(b2e, (L,))
    edge_out_p, zpart, spart = _sc_call(ab_lo, ab_hi, z, idxp, srows_p,
                                        w2e, b2e16)
    edge_out = edge_out_p.reshape(NW, EPAD)[:, :EPW]
    x_recon = _attr_call(zpart, spart, W1a[1:], W1a[0:1],
                         b1a.reshape(1, H), W2a, b2a.reshape(1, OUT))
    return (x_recon, edge_out.reshape(E, 1))


# speedup vs baseline: 1.1286x; 1.0516x over previous
import jax
import jax.numpy as jnp
from jax import lax
from jax.experimental import pallas as pl
from jax.experimental.pallas import tpu as pltpu
from jax.experimental.pallas import tpu_sc as plsc

N = 10000
E = 320000
D = 128
H = 256
OUT = 128
HH = H // 2

NC = 2
NS = 16
NW = NC * NS
EPW = E // NW
CHUNK = 48
NCHUNK = 210
EPAD = NCHUNK * CHUNK
PAD = EPAD - EPW
NROW = N + 8
RPT = 624
L = 16

RCOPIES = tuple((k * CHUNK, CHUNK) for k in range(13))


def _sc_body(ab_lo, ab_hi, z, idxp, srows, w2e, b2e,
             edge_out, zpart, spart,
             idxv, h0, h1, zrow, srow, out_v, w2e_v, b2e_v,
             zacc, sacc, sem0, sem1):
    cid = lax.axis_index("c")
    sid = lax.axis_index("s")
    wid = cid * NS + sid
    lanes = lax.iota(jnp.int32, L)
    zero16 = jnp.zeros((L,), jnp.float32)

    def zr(i, c):
        for j in range(D // L):
            zrow[i, pl.ds(j * L, L)] = zero16
        srow[i, :] = zero16
        return c
    lax.fori_loop(0, CHUNK, zr, 0)
    for off, sz in RCOPIES:
        pltpu.sync_copy(zrow.at[pl.ds(0, sz)],
                        zacc.at[pl.ds(sid * RPT + off, sz)])
        pltpu.sync_copy(srow.at[pl.ds(0, sz)],
                        sacc.at[pl.ds(sid * RPT + off, sz)])

    @pl.when(sid == NS - 1)
    def _zero_rem():
        pltpu.sync_copy(zrow.at[pl.ds(0, NROW - NS * RPT)],
                        zacc.at[pl.ds(NS * RPT, NROW - NS * RPT)])
        pltpu.sync_copy(srow.at[pl.ds(0, NROW - NS * RPT)],
                        sacc.at[pl.ds(NS * RPT, NROW - NS * RPT)])

    pltpu.sync_copy(w2e, w2e_v)
    pltpu.sync_copy(b2e, b2e_v)

    plsc.subcore_barrier()

    def half_reduce(buf, wbase, first):
        for g in range(CHUNK // L):
            e_vec = lanes + g * L

            def hchunk(hc, acc):
                w16 = w2e_v[pl.ds(wbase + hc * L, L)]
                for j in range(L):
                    h_vec = jnp.full((L,), hc * L + j, jnp.int32)
                    a = plsc.load_gather(buf, [e_vec, h_vec])
                    b = plsc.load_gather(buf, [e_vec + CHUNK, h_vec])
                    acc = acc + jnp.maximum(a + b, 0.0) * w16[j]
                return acc
            acc = lax.fori_loop(0, HH // L, hchunk, zero16)
            if first:
                out_v[pl.ds(g * L, L)] = acc + b2e_v[:]
            else:
                out_v[pl.ds(g * L, L)] = out_v[pl.ds(g * L, L)] + acc

    pltpu.sync_copy(idxp.at[pl.ds(wid * NCHUNK * 4 * CHUNK, 4 * CHUNK)], idxv)
    pltpu.async_copy(ab_lo.at[idxv.at[pl.ds(0, 2 * CHUNK)]], h0, sem0)

    def chunk_body(ci, c):
        eb = (wid * NCHUNK + ci) * CHUNK
        pltpu.make_async_copy(ab_lo.at[idxv.at[pl.ds(0, 2 * CHUNK)]],
                              h0, sem0).wait()
        pltpu.async_copy(ab_hi.at[idxv.at[pl.ds(0, 2 * CHUNK)]], h1, sem1)
        pltpu.sync_copy(srows.at[pl.ds(eb, CHUNK)], srow)
        pltpu.sync_copy(z.at[idxv.at[pl.ds(2 * CHUNK, CHUNK)]], zrow)
        pltpu.sync_copy(zrow, zacc.at[idxv.at[pl.ds(3 * CHUNK, CHUNK)]],
                        add=True)
        pltpu.sync_copy(srow, sacc.at[idxv.at[pl.ds(3 * CHUNK, CHUNK)]],
                        add=True)
        half_reduce(h0, 0, True)
        pltpu.make_async_copy(ab_hi.at[idxv.at[pl.ds(0, 2 * CHUNK)]],
                              h1, sem1).wait()

        @pl.when(ci < NCHUNK - 1)
        def _prefetch_next():
            ib = (wid * NCHUNK + ci + 1) * (4 * CHUNK)
            pltpu.sync_copy(idxp.at[pl.ds(ib, 4 * CHUNK)], idxv)
            pltpu.async_copy(ab_lo.at[idxv.at[pl.ds(0, 2 * CHUNK)]], h0, sem0)

        half_reduce(h1, HH, False)
        pltpu.sync_copy(out_v, edge_out.at[pl.ds(eb, CHUNK)])
        return c

    lax.fori_loop(0, NCHUNK, chunk_body, 0)

    plsc.subcore_barrier()

    def _dump(rbase, rn_static):
        pltpu.sync_copy(zacc.at[pl.ds(rbase, rn_static)],
                        zrow.at[pl.ds(0, rn_static)])
        pltpu.sync_copy(zrow.at[pl.ds(0, rn_static)],
                        zpart.at[cid, pl.ds(rbase, rn_static)])
        pltpu.sync_copy(sacc.at[pl.ds(rbase, rn_static)],
                        srow.at[pl.ds(0, rn_static)])
        pltpu.sync_copy(srow.at[pl.ds(0, rn_static)],
                        spart.at[cid, pl.ds(rbase, rn_static)])

    for off, sz in RCOPIES:
        _dump(sid * RPT + off, sz)

    @pl.when(sid == NS - 1)
    def _dump_rem():
        _dump(NS * RPT, N - NS * RPT)


_sc_call = pl.kernel(
    _sc_body,
    out_type=(
        jax.ShapeDtypeStruct((NW * EPAD,), jnp.float32),
        jax.ShapeDtypeStruct((NC, N, D), jnp.float32),
        jax.ShapeDtypeStruct((NC, N, L), jnp.float32),
    ),
    mesh=plsc.VectorSubcoreMesh(core_axis_name="c", subcore_axis_name="s"),
    compiler_params=pltpu.CompilerParams(needs_layout_passes=False,
                                         use_tc_tiling_on_sc=False),
    scratch_types=[
        pltpu.VMEM((4 * CHUNK,), jnp.int32),
        pltpu.VMEM((2 * CHUNK, HH), jnp.float32),
        pltpu.VMEM((2 * CHUNK, HH), jnp.float32),
        pltpu.VMEM((CHUNK, D), jnp.float32),
        pltpu.VMEM((CHUNK, L), jnp.float32),
        pltpu.VMEM((CHUNK,), jnp.float32),
        pltpu.VMEM((H,), jnp.float32),
        pltpu.VMEM((L,), jnp.float32),
        pltpu.VMEM_SHARED((NROW, D), jnp.float32),
        pltpu.VMEM_SHARED((NROW, L), jnp.float32),
        pltpu.SemaphoreType.DMA,
        pltpu.SemaphoreType.DMA,
    ],
)


def _mm_bias_body(x_ref, w_ref, b_ref, o_ref):
    o_ref[...] = (jnp.dot(x_ref[...], w_ref[...],
                          preferred_element_type=jnp.float32) + b_ref[...])


def _mm_bias(x, w, b, bn=1000):
    n, k = x.shape
    m = w.shape[1]
    return pl.pallas_call(
        _mm_bias_body,
        grid=(n // bn,),
        in_specs=[
            pl.BlockSpec((bn, k), lambda i: (i, 0)),
            pl.BlockSpec((k, m), lambda i: (0, 0)),
            pl.BlockSpec((1, m), lambda i: (0, 0)),
        ],
        out_specs=pl.BlockSpec((bn, m), lambda i: (i, 0)),
        out_shape=jax.ShapeDtypeStruct((n, m), jnp.float32),
    )(x, w, b.reshape(1, m))


def _attr_body(zp_ref, sp_ref, w1az_ref, w1aa_ref, b1a_ref, w2a_ref, b2a_ref,
               o_ref):
    zsum = zp_ref[0] + zp_ref[1]
    ssum = sp_ref[0] + sp_ref[1]
    cnt = jnp.maximum(ssum[:, 1:2], 1.0)
    attr_mean = ssum[:, 0:1] / cnt
    zmean = zsum / cnt
    h = jnp.dot(zmean, w1az_ref[...], preferred_element_type=jnp.float32)
    h = jnp.maximum(h + attr_mean * w1aa_ref[...] + b1a_ref[...], 0.0)
    o_ref[...] = (jnp.dot(h, w2a_ref[...],
                          preferred_element_type=jnp.float32) + b2a_ref[...])


def _attr_call(zpart, spart, w1az, w1aa, b1a, w2a, b2a, bn=1000):
    return pl.pallas_call(
        _attr_body,
        grid=(N // bn,),
        in_specs=[
            pl.BlockSpec((NC, bn, D), lambda i: (0, i, 0)),
            pl.BlockSpec((NC, bn, L), lambda i: (0, i, 0)),
            pl.BlockSpec((D, H), lambda i: (0, 0)),
            pl.BlockSpec((1, H), lambda i: (0, 0)),
            pl.BlockSpec((1, H), lambda i: (0, 0)),
            pl.BlockSpec((H, OUT), lambda i: (0, 0)),
            pl.BlockSpec((1, OUT), lambda i: (0, 0)),
        ],
        out_specs=pl.BlockSpec((bn, OUT), lambda i: (i, 0)),
        out_shape=jax.ShapeDtypeStruct((N, OUT), jnp.float32),
    )(zpart, spart, w1az, w1aa, b1a, w2a, b2a)


def _pad_tile(x, fill):
    x2 = x.reshape(NW, EPW)
    pad = jnp.full((NW, PAD), fill, x.dtype)
    return jnp.concatenate([x2, pad], axis=1)


def kernel(z, edge_index, edge_attr, W1a, b1a, W2a, b2a, W1e, b1e, W2e, b2e):
    src = edge_index[0].astype(jnp.int32)
    dst = edge_index[1].astype(jnp.int32)
    attr = edge_attr[:, 0]
    ga_p = _pad_tile(src * 2, 0).reshape(NW, NCHUNK, CHUNK)
    gb_p = _pad_tile(dst * 2 + 1, 1).reshape(NW, NCHUNK, CHUNK)
    src_p = _pad_tile(src, 0).reshape(NW, NCHUNK, CHUNK)
    dst_p = _pad_tile(dst, N).reshape(NW, NCHUNK, CHUNK)
    idxp = jnp.stack([ga_p, gb_p, src_p, dst_p], axis=2).reshape(-1)
    srows = jnp.concatenate(
        [attr[:, None], jnp.ones((E, 1), jnp.float32),
         jnp.zeros((E, L - 2), jnp.float32)], axis=1)
    srows_p = jnp.concatenate(
        [srows.reshape(NW, EPW, L), jnp.zeros((NW, PAD, L), jnp.float32)],
        axis=1).reshape(NW * EPAD, L)
    wcat = jnp.concatenate(
        [W1e[:D, :HH], W1e[D:, :HH], W1e[:D, HH:], W1e[D:, HH:]], axis=1)
    bcat = jnp.concatenate(
        [b1e[:HH], jnp.zeros((HH,), b1e.dtype),
         b1e[HH:], jnp.zeros((HH,), b1e.dtype)])
    ab = _mm_bias(z, wcat, bcat)
    ab_lo = ab[:, :H].reshape(2 * N, HH)
    ab_hi = ab[:, H:].reshape(2 * N, HH)
    w2e = W2e[:, 0]
    b2e16 = jnp.broadcast_to(b2e, (L,))
    edge_out_p, zpart, spart = _sc_call(ab_lo, ab_hi, z, idxp, srows_p,
                                        w2e, b2e16)
    edge_out = edge_out_p.reshape(NW, EPAD)[:, :EPW]
    x_recon = _attr_call(zpart, spart, W1a[1:], W1a[0:1],
                         b1a.reshape(1, H), W2a, b2a.reshape(1, OUT))
    return (x_recon, edge_out.reshape(E, 1))
